# baseline (speedup 1.0000x reference)
import jax
import jax.numpy as jnp
from jax.experimental import pallas as pl
from jax.experimental.pallas import tpu as pltpu

B = 32
S = 1
HIDDEN = 2048
DFF = 768
E = 64
TOPK = 8
T = B * S


def _moe_kernel(x_ref, gate_w_ref, wg_ref, wu_ref, wd_ref, out_ref,
                rw_ref, acc_ref, glu_ref):
    i = pl.program_id(0)

    @pl.when(i == 0)
    def _router():
        x = x_ref[...]
        logits = jax.lax.dot_general(
            x, gate_w_ref[...],
            (((1,), (1,)), ((), ())),
            preferred_element_type=jnp.float32)
        col = jax.lax.broadcasted_iota(jnp.int32, (T, E), 1)
        neg_inf = jnp.float32(-jnp.inf)
        cur = logits
        sel = jnp.zeros((T, E), dtype=jnp.bool_)
        for _ in range(TOPK):
            mx = jnp.max(cur, axis=1, keepdims=True)
            at_max = cur == mx
            first = jnp.min(jnp.where(at_max, col, E), axis=1, keepdims=True)
            pick = col == first
            sel = jnp.logical_or(sel, pick)
            cur = jnp.where(pick, neg_inf, cur)
        z = jnp.where(sel, logits, neg_inf)
        zmax = jnp.max(z, axis=1, keepdims=True)
        p = jnp.where(sel, jnp.exp(z - zmax), 0.0)
        rw_ref[...] = p / jnp.sum(p, axis=1, keepdims=True)
        acc_ref[...] = jnp.zeros_like(acc_ref)

    @pl.when(i > 0)
    def _down_prev():
        o = jax.lax.dot_general(glu_ref[...], wd_ref[0],
                                (((1,), (1,)), ((), ())),
                                preferred_element_type=jnp.float32)
        rw = rw_ref[...]
        ecol = jax.lax.broadcasted_iota(jnp.int32, (T, E), 1)
        w_col = jnp.sum(jnp.where(ecol == i - 1, rw, 0.0),
                        axis=1, keepdims=True)
        acc_ref[...] += w_col * o

    @pl.when(i < E)
    def _gate_up():
        x = x_ref[...]
        g = jax.lax.dot_general(x, wg_ref[0], (((1,), (1,)), ((), ())),
                                preferred_element_type=jnp.float32)
        u = jax.lax.dot_general(x, wu_ref[0], (((1,), (1,)), ((), ())),
                                preferred_element_type=jnp.float32)
        glu_ref[...] = g * jax.nn.sigmoid(g) * u

    @pl.when(i == E)
    def _write():
        out_ref[...] = acc_ref[...]


def kernel(hidden_states, gate_w, w_gate, w_up, w_down):
    x = hidden_states.reshape(T, HIDDEN)
    last = E - 1
    out = pl.pallas_call(
        _moe_kernel,
        grid=(E + 1,),
        in_specs=[
            pl.BlockSpec((T, HIDDEN), lambda i: (0, 0)),
            pl.BlockSpec((E, HIDDEN), lambda i: (0, 0)),
            pl.BlockSpec((1, DFF, HIDDEN), lambda i: (jnp.minimum(i, last), 0, 0)),
            pl.BlockSpec((1, DFF, HIDDEN), lambda i: (jnp.minimum(i, last), 0, 0)),
            pl.BlockSpec((1, HIDDEN, DFF), lambda i: (jnp.maximum(i - 1, 0), 0, 0)),
        ],
        out_specs=pl.BlockSpec((T, HIDDEN), lambda i: (0, 0)),
        out_shape=jax.ShapeDtypeStruct((T, HIDDEN), jnp.float32),
        scratch_shapes=[
            pltpu.VMEM((T, E), jnp.float32),
            pltpu.VMEM((T, HIDDEN), jnp.float32),
            pltpu.VMEM((T, DFF), jnp.float32),
        ],
    )(x, gate_w, w_gate, w_up, w_down)
    return out.reshape(B, S, HIDDEN)
